# baseline (speedup 1.0000x reference)
import functools

import jax
import jax.numpy as jnp
from jax import lax
from jax.experimental import pallas as pl
from jax.experimental.pallas import tpu as pltpu
from jax.experimental.pallas import tpu_sc as plsc

B, L = 16, 64
D, H = 128, 256
NE, NA = 34, 36
BL = B * L
LANES = 128
F32 = jnp.float32


def _make_sc_gather(V):
  info = plsc.get_sparse_core_info()
  NW = info.num_cores * info.num_subcores
  b_per_w = BL // NW
  mesh = plsc.VectorSubcoreMesh(core_axis_name="c", subcore_axis_name="s")

  @functools.partial(
      pl.kernel, mesh=mesh,
      out_type=jax.ShapeDtypeStruct((BL, D), F32),
      scratch_types=[
          pltpu.VMEM((b_per_w,), jnp.int32),
          pltpu.VMEM((b_per_w, D), F32),
          pltpu.SemaphoreType.DMA,
      ],
  )
  def gather_k(table_hbm, idx_hbm, out_hbm, idx_v, rows_v, sem):
    wid = lax.axis_index("s") * info.num_cores + lax.axis_index("c")
    base = wid * b_per_w
    pltpu.sync_copy(idx_hbm.at[pl.ds(base, b_per_w)], idx_v)
    pltpu.async_copy(table_hbm.at[idx_v], rows_v, sem).wait()
    pltpu.sync_copy(rows_v, out_hbm.at[pl.ds(base, b_per_w)])

  return gather_k


_SC_GATHER = None


def _sc_gather(emb, ids):
  global _SC_GATHER
  if _SC_GATHER is None:
    _SC_GATHER = _make_sc_gather(emb.shape[0])
  return _SC_GATHER(emb, ids)


def _dot_t(a, b):
  return lax.dot_general(a, b, (((1,), (1,)), ((), ())),
                         preferred_element_type=F32)


def _lstm_body(x_ref, wi_ref, whf, whb, bif, bhf, bib, bhb, wheads_ref,
               be, ev_ref, meta_ref, hsc_ref, trig_ref, gi_s, hs_s):
  x = x_ref[...]
  gi_s[...] = _dot_t(x, wi_ref[...])
  whf_v = whf[...]
  whb_v = whb[...]
  bif_v, bhf_v, bib_v, bhb_v = bif[...], bhf[...], bib[...], bhb[...]

  def step(t, carry):
    hf, cf, hb, cb = carry
    gf = ((gi_s[pl.ds(t * B, B), 0:4 * H]
           + _dot_t(hf, whf_v)) + bif_v) + bhf_v
    cf = jax.nn.sigmoid(gf[:, H:2 * H]) * cf + \
        jax.nn.sigmoid(gf[:, :H]) * jnp.tanh(gf[:, 2 * H:3 * H])
    hf = jax.nn.sigmoid(gf[:, 3 * H:]) * jnp.tanh(cf)
    hs_s[pl.ds(t * B, B), 0:H] = hf
    tb = (L - 1) - t
    gb = ((gi_s[pl.ds(tb * B, B), 4 * H:8 * H]
           + _dot_t(hb, whb_v)) + bib_v) + bhb_v
    cb = jax.nn.sigmoid(gb[:, H:2 * H]) * cb + \
        jax.nn.sigmoid(gb[:, :H]) * jnp.tanh(gb[:, 2 * H:3 * H])
    hb = jax.nn.sigmoid(gb[:, 3 * H:]) * jnp.tanh(cb)
    hs_s[pl.ds(tb * B, B), H:2 * H] = hb
    return hf, cf, hb, cb

  z = jnp.zeros((B, H), F32)
  lax.fori_loop(0, L, step, (z, z, z, z))
  hs = hs_s[...]
  heads = _dot_t(hs, wheads_ref[...])
  ev = heads[:, 0:LANES] + be[...]
  ev_ref[...] = ev[:, :NE]
  lane = lax.broadcasted_iota(jnp.int32, (BL, LANES), 1)
  evm = jnp.where(lane < NE, ev, -jnp.inf)
  mx = jnp.max(evm, axis=1, keepdims=True)
  idx = jnp.min(jnp.where(evm == mx, lane, LANES), axis=1, keepdims=True)
  mb = idx > 0
  colp = jnp.clip(idx - 1, 0, NE - 2) + (NA - 1)
  meta_ref[...] = (mb & ((lane == colp) | (lane == 120))).astype(F32)
  hsc_ref[...] = heads[:, LANES:2 * LANES]
  trig_ref[...] = heads[:, 2 * LANES:3 * LANES]


_LSTM_KW = dict(
    out_shape=[
        jax.ShapeDtypeStruct((BL, NE), F32),
        jax.ShapeDtypeStruct((BL, LANES), F32),
        jax.ShapeDtypeStruct((BL, LANES), F32),
        jax.ShapeDtypeStruct((BL, LANES), F32),
    ],
    scratch_shapes=[
        pltpu.VMEM((BL, 8 * H), F32),
        pltpu.VMEM((BL, 2 * H), F32),
    ],
)


KSTEP = 8


def _dec_body(hsc_ref, trig_ref, pay_ref, wg_ref, ba_ref, out_ref, g_s):
  i = pl.program_id(0)

  @pl.when(i == 0)
  def _init():
    g_s[...] = jnp.zeros((B, L, LANES), F32)

  g = g_s[...]
  hsc = hsc_ref[...]
  ba = ba_ref[...].reshape(1, 1, LANES)
  wg = wg_ref[...]
  lane = lax.broadcasted_iota(jnp.int32, (B, L, LANES), 2)
  for k in range(KSTEP):
    gc = jnp.dot(g.reshape(BL, LANES), wg,
                 preferred_element_type=F32).reshape(B, L, LANES)
    trig = trig_ref[k].reshape(B, 1, LANES)
    pay = pay_ref[k].reshape(B, 1, LANES)
    logits = ((hsc + trig) + gc) + ba
    out_ref[:, k] = logits[:, :, :NA]
    mxv = jnp.max(logits, axis=2, keepdims=True)
    ap = jnp.min(jnp.where(logits == mxv, lane, LANES), axis=2, keepdims=True)
    upd = (ap > 0) & (
        ((pay[:, :, 120:121] > 0.5) & (lane == (ap - 1))) | (pay > 0.5))
    g = jnp.where(upd, 1.0, g)
  g_s[...] = g


_DEC_KW = dict(
    grid=(L // KSTEP,),
    in_specs=[
        pl.BlockSpec((B, L, LANES), lambda i: (0, 0, 0)),
        pl.BlockSpec((KSTEP, B, 1, LANES), lambda i: (i, 0, 0, 0)),
        pl.BlockSpec((KSTEP, B, 1, LANES), lambda i: (i, 0, 0, 0)),
        pl.BlockSpec((LANES, LANES), lambda i: (0, 0)),
        pl.BlockSpec((1, LANES), lambda i: (0, 0)),
    ],
    out_specs=pl.BlockSpec((B, KSTEP, L, NA), lambda i: (0, i, 0, 0)),
    out_shape=jax.ShapeDtypeStruct((B, L, L, NA), F32),
    scratch_shapes=[pltpu.VMEM((B, L, LANES), F32)],
    compiler_params=pltpu.CompilerParams(dimension_semantics=("arbitrary",)),
)


def _pad_cols(w, cols):
  return jnp.zeros((w.shape[0], cols), F32).at[:, :w.shape[1]].set(w)


def kernel(input_ids, emb, W_ih_f, W_hh_f, b_ih_f, b_hh_f, W_ih_b, W_hh_b,
           b_ih_b, b_hh_b, W_e, b_e, W_a, b_a):
  ids_t = input_ids.astype(jnp.int32).T.reshape(BL)
  x = _sc_gather(emb, ids_t)

  bif = b_ih_f.reshape(1, 4 * H)
  bhf = b_hh_f.reshape(1, 4 * H)
  bib = b_ih_b.reshape(1, 4 * H)
  bhb = b_hh_b.reshape(1, 4 * H)
  wi_cat = jnp.concatenate([W_ih_f, W_ih_b], axis=0)
  zrow = jnp.zeros((LANES - NE, 2 * H), F32)
  zrow2 = jnp.zeros((LANES - NA, 2 * H), F32)
  wheads = jnp.concatenate([
      W_e, zrow,
      W_a[:, :2 * H], zrow2,
      W_a[:, 2 * H:4 * H], zrow2], axis=0)
  be = _pad_cols(b_e.reshape(1, NE), LANES)
  wg = jnp.zeros((LANES, LANES), F32)
  wg = wg.at[:NA - 1, :NA].set(W_a[:, 4 * H:4 * H + NA - 1].T)
  wg = wg.at[NA - 1:NA - 1 + NE - 1, :NA].set(W_a[:, 4 * H + NA - 1:].T)

  ev_t, pay_t, hsc_t, trig_t = pl.pallas_call(_lstm_body, **_LSTM_KW)(
      x, wi_cat, W_hh_f, W_hh_b, bif, bhf, bib, bhb, wheads, be)

  event_logits = ev_t.reshape(L, B, NE).transpose(1, 0, 2)
  hsc3 = hsc_t.reshape(L, B, LANES).transpose(1, 0, 2)
  trig4 = trig_t.reshape(L, B, 1, LANES)
  pay4 = pay_t.reshape(L, B, 1, LANES)
  ba_dec = jnp.concatenate(
      [b_a.astype(F32), jnp.full((LANES - NA,), -1e30, F32)]).reshape(1, LANES)

  arg_logits = pl.pallas_call(_dec_body, **_DEC_KW)(hsc3, trig4, pay4, wg, ba_dec)
  return event_logits, arg_logits


# speedup vs baseline: 1.2280x; 1.2280x over previous
import functools

import jax
import jax.numpy as jnp
from jax import lax
from jax.experimental import pallas as pl
from jax.experimental.pallas import tpu as pltpu
from jax.experimental.pallas import tpu_sc as plsc

B, L = 16, 64
D, H = 128, 256
NE, NA = 34, 36
BL = B * L
LANES = 128
F32 = jnp.float32


def _make_sc_gather(V):
  info = plsc.get_sparse_core_info()
  NW = info.num_cores * info.num_subcores
  b_per_w = BL // NW
  mesh = plsc.VectorSubcoreMesh(core_axis_name="c", subcore_axis_name="s")

  @functools.partial(
      pl.kernel, mesh=mesh,
      out_type=jax.ShapeDtypeStruct((BL, D), F32),
      scratch_types=[
          pltpu.VMEM((b_per_w,), jnp.int32),
          pltpu.VMEM((b_per_w, D), F32),
          pltpu.SemaphoreType.DMA,
      ],
  )
  def gather_k(table_hbm, idx_hbm, out_hbm, idx_v, rows_v, sem):
    wid = lax.axis_index("s") * info.num_cores + lax.axis_index("c")
    base = wid * b_per_w
    pltpu.sync_copy(idx_hbm.at[pl.ds(base, b_per_w)], idx_v)
    pltpu.async_copy(table_hbm.at[idx_v], rows_v, sem).wait()
    pltpu.sync_copy(rows_v, out_hbm.at[pl.ds(base, b_per_w)])

  return gather_k


_SC_GATHER = None


def _sc_gather(emb, ids):
  global _SC_GATHER
  if _SC_GATHER is None:
    _SC_GATHER = _make_sc_gather(emb.shape[0])
  return _SC_GATHER(emb, ids)


def _lstm_body(x_ref, wi_ref, whf, whb, bif, bhf, bib, bhb, wheads_ref,
               be, ev_ref, meta_ref, hsc_ref, trig_ref, gi_s, hs_s):
  x = x_ref[...]
  gi_s[...] = jnp.dot(x, wi_ref[...], preferred_element_type=F32)
  whf_v = whf[...]
  whb_v = whb[...]
  bif_v, bhf_v, bib_v, bhb_v = bif[...], bhf[...], bib[...], bhb[...]

  def step(t, carry):
    hf, cf, hb, cb = carry
    gf = ((gi_s[pl.ds(t * B, B), 0:4 * H]
           + jnp.dot(hf, whf_v, preferred_element_type=F32)) + bif_v) + bhf_v
    cf = jax.nn.sigmoid(gf[:, H:2 * H]) * cf + \
        jax.nn.sigmoid(gf[:, :H]) * jnp.tanh(gf[:, 2 * H:3 * H])
    hf = jax.nn.sigmoid(gf[:, 3 * H:]) * jnp.tanh(cf)
    hs_s[pl.ds(t * B, B), 0:H] = hf
    tb = (L - 1) - t
    gb = ((gi_s[pl.ds(tb * B, B), 4 * H:8 * H]
           + jnp.dot(hb, whb_v, preferred_element_type=F32)) + bib_v) + bhb_v
    cb = jax.nn.sigmoid(gb[:, H:2 * H]) * cb + \
        jax.nn.sigmoid(gb[:, :H]) * jnp.tanh(gb[:, 2 * H:3 * H])
    hb = jax.nn.sigmoid(gb[:, 3 * H:]) * jnp.tanh(cb)
    hs_s[pl.ds(tb * B, B), H:2 * H] = hb
    return hf, cf, hb, cb

  z = jnp.zeros((B, H), F32)
  lax.fori_loop(0, L, step, (z, z, z, z))
  hs = hs_s[...]
  heads = jnp.dot(hs, wheads_ref[...], preferred_element_type=F32)
  ev = heads[:, 0:LANES] + be[...]
  ev_ref[...] = ev[:, :NE]
  lane = lax.broadcasted_iota(jnp.int32, (BL, LANES), 1)
  evm = jnp.where(lane < NE, ev, -jnp.inf)
  mx = jnp.max(evm, axis=1, keepdims=True)
  idx = jnp.min(jnp.where(evm == mx, lane, LANES), axis=1, keepdims=True)
  mb = idx > 0
  colp = jnp.clip(idx - 1, 0, NE - 2) + (NA - 1)
  meta_ref[...] = (mb & ((lane == colp) | (lane == 120))).astype(F32)
  hsc_ref[...] = heads[:, LANES:2 * LANES]
  trig_ref[...] = heads[:, 2 * LANES:3 * LANES]


_LSTM_KW = dict(
    out_shape=[
        jax.ShapeDtypeStruct((BL, NE), F32),
        jax.ShapeDtypeStruct((BL, LANES), F32),
        jax.ShapeDtypeStruct((BL, LANES), F32),
        jax.ShapeDtypeStruct((BL, LANES), F32),
    ],
    scratch_shapes=[
        pltpu.VMEM((BL, 8 * H), F32),
        pltpu.VMEM((BL, 2 * H), F32),
    ],
)


KSTEP = 8


def _dec_body(hsc_ref, trig_ref, pay_ref, wg_ref, ba_ref, out_ref, g_s):
  i = pl.program_id(0)

  @pl.when(i == 0)
  def _init():
    g_s[...] = jnp.zeros((B, L, LANES), F32)

  g = g_s[...]
  hsc = hsc_ref[...]
  ba = ba_ref[...].reshape(1, 1, LANES)
  wg = wg_ref[...]
  lane_f = lax.broadcasted_iota(jnp.int32, (B, L, LANES), 2).astype(F32)
  for k in range(KSTEP):
    gc = jnp.dot(g.reshape(BL, LANES), wg,
                 preferred_element_type=F32).reshape(B, L, LANES)
    trig = trig_ref[k].reshape(B, 1, LANES)
    pay = pay_ref[k].reshape(B, 1, LANES)
    logits = ((hsc + trig) + gc) + ba
    out_ref[:, k] = logits[:, :, :NA]
    mxv = jnp.max(logits, axis=2, keepdims=True)
    apf = jnp.min(jnp.where(logits == mxv, lane_f, 128.0),
                  axis=2, keepdims=True)
    upd = (apf > 0.5) & (
        ((pay[:, :, 120:121] > 0.5) & (lane_f == (apf - 1.0))) | (pay > 0.5))
    g = jnp.where(upd, 1.0, g)
  g_s[...] = g


_DEC_KW = dict(
    grid=(L // KSTEP,),
    in_specs=[
        pl.BlockSpec((B, L, LANES), lambda i: (0, 0, 0)),
        pl.BlockSpec((KSTEP, B, 1, LANES), lambda i: (i, 0, 0, 0)),
        pl.BlockSpec((KSTEP, B, 1, LANES), lambda i: (i, 0, 0, 0)),
        pl.BlockSpec((LANES, LANES), lambda i: (0, 0)),
        pl.BlockSpec((1, LANES), lambda i: (0, 0)),
    ],
    out_specs=pl.BlockSpec((B, KSTEP, L, NA), lambda i: (0, i, 0, 0)),
    out_shape=jax.ShapeDtypeStruct((B, L, L, NA), F32),
    scratch_shapes=[pltpu.VMEM((B, L, LANES), F32)],
    compiler_params=pltpu.CompilerParams(dimension_semantics=("arbitrary",)),
)


def _pad_cols(w, cols):
  return jnp.zeros((w.shape[0], cols), F32).at[:, :w.shape[1]].set(w)


def kernel(input_ids, emb, W_ih_f, W_hh_f, b_ih_f, b_hh_f, W_ih_b, W_hh_b,
           b_ih_b, b_hh_b, W_e, b_e, W_a, b_a):
  ids_t = input_ids.astype(jnp.int32).T.reshape(BL)
  x = _sc_gather(emb, ids_t)

  bif = b_ih_f.reshape(1, 4 * H)
  bhf = b_hh_f.reshape(1, 4 * H)
  bib = b_ih_b.reshape(1, 4 * H)
  bhb = b_hh_b.reshape(1, 4 * H)
  wi_cat = jnp.concatenate([W_ih_f, W_ih_b], axis=0).T
  zrow = jnp.zeros((LANES - NE, 2 * H), F32)
  zrow2 = jnp.zeros((LANES - NA, 2 * H), F32)
  wheads = jnp.concatenate([
      W_e, zrow,
      W_a[:, :2 * H], zrow2,
      W_a[:, 2 * H:4 * H], zrow2], axis=0).T
  be = _pad_cols(b_e.reshape(1, NE), LANES)
  wg = jnp.zeros((LANES, LANES), F32)
  wg = wg.at[:NA - 1, :NA].set(W_a[:, 4 * H:4 * H + NA - 1].T)
  wg = wg.at[NA - 1:NA - 1 + NE - 1, :NA].set(W_a[:, 4 * H + NA - 1:].T)

  ev_t, pay_t, hsc_t, trig_t = pl.pallas_call(_lstm_body, **_LSTM_KW)(
      x, wi_cat, W_hh_f.T, W_hh_b.T, bif, bhf, bib, bhb, wheads, be)

  event_logits = ev_t.reshape(L, B, NE).transpose(1, 0, 2)
  hsc3 = hsc_t.reshape(L, B, LANES).transpose(1, 0, 2)
  trig4 = trig_t.reshape(L, B, 1, LANES)
  pay4 = pay_t.reshape(L, B, 1, LANES)
  ba_dec = jnp.concatenate(
      [b_a.astype(F32), jnp.full((LANES - NA,), -1e30, F32)]).reshape(1, LANES)

  arg_logits = pl.pallas_call(_dec_body, **_DEC_KW)(hsc3, trig4, pay4, wg, ba_dec)
  return event_logits, arg_logits
